# Initial kernel scaffold; baseline (speedup 1.0000x reference)
import jax
import jax.numpy as jnp
from jax.experimental import pallas as pl


def kernel(x, edge_index, W1l, b1, W1r, W2l, b2, W2r, W3l, b3, W3r):
    raise NotImplementedError("write your pallas kernel here")


# speedup vs baseline: 32.9284x; 32.9284x over previous
import functools

import jax
import jax.numpy as jnp
from jax import lax
from jax.experimental import pallas as pl
from jax.experimental.pallas import tpu as pltpu
from jax.experimental.pallas import tpu_sc as plsc

NN = 100000
NE = 3200000
LANE = 128
KG = 8
NC, NS = 2, 16
NW = NC * NS
RPT = 784
NROWS = RPT * NW
EP = NROWS * LANE
N_ACC = 102400
ZR = N_ACC // NS


def _seg_sum(width):
    mesh = plsc.VectorSubcoreMesh(core_axis_name="c", subcore_axis_name="s")

    @functools.partial(
        pl.kernel,
        out_type=jax.ShapeDtypeStruct((NC, N_ACC, width), jnp.float32),
        mesh=mesh,
        scratch_types=[
            pltpu.VMEM_SHARED((N_ACC, width), jnp.float32),
            pltpu.VMEM((KG, LANE), jnp.int32),
            pltpu.VMEM((KG, LANE), jnp.int32),
            pltpu.VMEM((KG, LANE, width), jnp.float32),
            pltpu.SemaphoreType.DMA,
            pltpu.SemaphoreType.DMA,
        ],
        compiler_params=pltpu.CompilerParams(use_tc_tiling_on_sc=False),
    )
    def seg(src_hbm, dst_hbm, table_hbm, zeros_hbm, out_hbm,
            accum, sbuf, dbuf, rbuf, gsem, ssem):
        c = lax.axis_index("c")
        s = lax.axis_index("s")
        tid = s * NC + c
        pltpu.sync_copy(zeros_hbm.at[pl.ds(s * ZR, ZR)], accum.at[pl.ds(s * ZR, ZR)])
        plsc.subcore_barrier()

        row_base = tid * RPT

        def chunk(m, carry):
            r0 = row_base + m * KG
            pltpu.sync_copy(src_hbm.at[pl.ds(r0, KG)], sbuf)
            pltpu.sync_copy(dst_hbm.at[pl.ds(r0, KG)], dbuf)
            gs = [pltpu.async_copy(table_hbm.at[sbuf.at[j]], rbuf.at[j], gsem)
                  for j in range(KG)]
            for g in gs:
                g.wait()
            ss = [pltpu.async_copy(rbuf.at[j], accum.at[dbuf.at[j]], ssem, add=True)
                  for j in range(KG)]
            for t in ss:
                t.wait()
            return carry

        lax.fori_loop(0, RPT // KG, chunk, 0)
        plsc.subcore_barrier()
        pltpu.sync_copy(accum.at[pl.ds(s * ZR, ZR)],
                        out_hbm.at[c, pl.ds(s * ZR, ZR)])

    return seg


_seg8 = _seg_sum(8)
_seg16 = _seg_sum(16)


_BN = 1000


def _full(shape):
    return pl.BlockSpec(shape, lambda i: tuple(0 for _ in shape))


def _dense1(P, x, W1l, b1, W1r):
    def body(p_ref, x_ref, wl, b, wr, h_ref, inv_ref):
        ps = p_ref[0] + p_ref[1]
        inv = 1.0 / jnp.maximum(ps[:, 3:4], 1.0)
        mean3 = ps[:, 0:3] * inv
        h = mean3 @ wl[...] + b[...] + x_ref[...] @ wr[...]
        h_ref[...] = jnp.maximum(h, 0.0)
        inv_ref[...] = inv

    return pl.pallas_call(
        body,
        grid=(NN // _BN,),
        in_specs=[
            pl.BlockSpec((NC, _BN, 8), lambda i: (0, i, 0)),
            pl.BlockSpec((_BN, 3), lambda i: (i, 0)),
            _full((3, 16)), _full((1, 16)), _full((3, 16)),
        ],
        out_specs=[pl.BlockSpec((_BN, 16), lambda i: (i, 0)),
                   pl.BlockSpec((_BN, 1), lambda i: (i, 0))],
        out_shape=[jax.ShapeDtypeStruct((NN, 16), jnp.float32),
                   jax.ShapeDtypeStruct((NN, 1), jnp.float32)],
    )(P, x, W1l, b1, W1r)


def _dense2(P, h1, inv, W2l, b2, W2r, W3l):
    def body(p_ref, h1_ref, inv_ref, wl, b, wr, w3, h_ref, y_ref):
        mean = (p_ref[0] + p_ref[1]) * inv_ref[...]
        h = mean @ wl[...] + b[...] + h1_ref[...] @ wr[...]
        h = jnp.maximum(h, 0.0)
        h_ref[...] = h
        y_ref[...] = h @ w3[...]

    return pl.pallas_call(
        body,
        grid=(NN // _BN,),
        in_specs=[
            pl.BlockSpec((NC, _BN, 16), lambda i: (0, i, 0)),
            pl.BlockSpec((_BN, 16), lambda i: (i, 0)),
            pl.BlockSpec((_BN, 1), lambda i: (i, 0)),
            _full((16, 16)), _full((1, 16)), _full((16, 16)), _full((16, 8)),
        ],
        out_specs=[pl.BlockSpec((_BN, 16), lambda i: (i, 0)),
                   pl.BlockSpec((_BN, 8), lambda i: (i, 0))],
        out_shape=[jax.ShapeDtypeStruct((NN, 16), jnp.float32),
                   jax.ShapeDtypeStruct((NN, 8), jnp.float32)],
    )(P, h1, inv, W2l, b2, W2r, W3l)


def _dense3(P, h2, inv, b3, W3r):
    def body(p_ref, h2_ref, inv_ref, b, wr, o_ref):
        mean_y = (p_ref[0, :, 0:1] + p_ref[1, :, 0:1]) * inv_ref[...]
        o_ref[...] = jax.nn.sigmoid(mean_y + b[...] + h2_ref[...] @ wr[...])

    return pl.pallas_call(
        body,
        grid=(NN // _BN,),
        in_specs=[
            pl.BlockSpec((NC, _BN, 8), lambda i: (0, i, 0)),
            pl.BlockSpec((_BN, 16), lambda i: (i, 0)),
            pl.BlockSpec((_BN, 1), lambda i: (i, 0)),
            _full((1, 1)), _full((16, 1)),
        ],
        out_specs=pl.BlockSpec((_BN, 1), lambda i: (i, 0)),
        out_shape=jax.ShapeDtypeStruct((NN, 1), jnp.float32),
    )(P, h2, inv, b3, W3r)


def kernel(x, edge_index, W1l, b1, W1r, W2l, b2, W2r, W3l, b3, W3r):
    pad = EP - NE
    src = jnp.concatenate([edge_index[0], jnp.zeros((pad,), jnp.int32)])
    dst = jnp.concatenate([edge_index[1], jnp.full((pad,), NN, jnp.int32)])
    src2d = src.reshape(NROWS, LANE)
    dst2d = dst.reshape(NROWS, LANE)

    xpad = jnp.concatenate(
        [x, jnp.ones((NN, 1), jnp.float32), jnp.zeros((NN, 4), jnp.float32)],
        axis=1)
    P1 = _seg8(src2d, dst2d, xpad, jnp.zeros((N_ACC, 8), jnp.float32))
    h1, inv = _dense1(P1, x, W1l, b1.reshape(1, 16), W1r)

    P2 = _seg16(src2d, dst2d, h1, jnp.zeros((N_ACC, 16), jnp.float32))
    W3p = jnp.pad(W3l, ((0, 0), (0, 7)))
    h2, y = _dense2(P2, h1, inv, W2l, b2.reshape(1, 16), W2r, W3p)

    P3 = _seg8(src2d, dst2d, y, jnp.zeros((N_ACC, 8), jnp.float32))
    return _dense3(P3, h2, inv, b3.reshape(1, 1), W3r)


# speedup vs baseline: 36.1642x; 1.0983x over previous
import functools

import jax
import jax.numpy as jnp
from jax import lax
from jax.experimental import pallas as pl
from jax.experimental.pallas import tpu as pltpu
from jax.experimental.pallas import tpu_sc as plsc

NN = 100000
NE = 3200000
LANE = 128
NC, NS = 2, 16
NW = NC * NS
RPT = 784
NROWS = RPT * NW
EP = NROWS * LANE
N_ACC = 102400
ZR = N_ACC // NS


def _seg_sum(width, kg):
    PAIR = 2 * kg
    assert RPT % PAIR == 0
    mesh = plsc.VectorSubcoreMesh(core_axis_name="c", subcore_axis_name="s")

    @functools.partial(
        pl.kernel,
        out_type=jax.ShapeDtypeStruct((NC, N_ACC, width), jnp.float32),
        mesh=mesh,
        scratch_types=[
            pltpu.VMEM_SHARED((N_ACC, width), jnp.float32),
            pltpu.VMEM((PAIR, LANE), jnp.int32),
            pltpu.VMEM((PAIR, LANE), jnp.int32),
            pltpu.VMEM((kg, LANE, width), jnp.float32),
            pltpu.VMEM((kg, LANE, width), jnp.float32),
            pltpu.SemaphoreType.DMA,
            pltpu.SemaphoreType.DMA,
        ],
        compiler_params=pltpu.CompilerParams(use_tc_tiling_on_sc=False),
    )
    def seg(src_hbm, dst_hbm, table_hbm, zeros_hbm, out_hbm,
            accum, sbuf, dbuf, rbufa, rbufb, gsem, ssem):
        c = lax.axis_index("c")
        s = lax.axis_index("s")
        tid = s * NC + c
        pltpu.sync_copy(zeros_hbm.at[pl.ds(s * ZR, ZR)], accum.at[pl.ds(s * ZR, ZR)])
        plsc.subcore_barrier()

        row_base = tid * RPT

        def chunk(m, carry):
            r0 = row_base + m * PAIR
            pltpu.sync_copy(src_hbm.at[pl.ds(r0, PAIR)], sbuf)
            pltpu.sync_copy(dst_hbm.at[pl.ds(r0, PAIR)], dbuf)
            ga = [pltpu.async_copy(table_hbm.at[sbuf.at[j]], rbufa.at[j], gsem)
                  for j in range(kg)]
            gb = [pltpu.async_copy(table_hbm.at[sbuf.at[kg + j]], rbufb.at[j], gsem)
                  for j in range(kg)]
            for g in ga:
                g.wait()
            sa = [pltpu.async_copy(rbufa.at[j], accum.at[dbuf.at[j]], ssem, add=True)
                  for j in range(kg)]
            for g in gb:
                g.wait()
            sb = [pltpu.async_copy(rbufb.at[j], accum.at[dbuf.at[kg + j]], ssem, add=True)
                  for j in range(kg)]
            for t in sa + sb:
                t.wait()
            return carry

        lax.fori_loop(0, RPT // PAIR, chunk, 0)
        plsc.subcore_barrier()
        pltpu.sync_copy(accum.at[pl.ds(s * ZR, ZR)],
                        out_hbm.at[c, pl.ds(s * ZR, ZR)])

    return seg


_seg8 = _seg_sum(8, 7)
_seg16 = _seg_sum(16, 4)


_BN = 1000


def _full(shape):
    return pl.BlockSpec(shape, lambda i: tuple(0 for _ in shape))


def _dense1(P, x, W1l, b1, W1r):
    def body(p_ref, x_ref, wl, b, wr, h_ref, inv_ref):
        ps = p_ref[0] + p_ref[1]
        inv = 1.0 / jnp.maximum(ps[:, 3:4], 1.0)
        mean3 = ps[:, 0:3] * inv
        h = mean3 @ wl[...] + b[...] + x_ref[...] @ wr[...]
        h_ref[...] = jnp.maximum(h, 0.0)
        inv_ref[...] = inv

    return pl.pallas_call(
        body,
        grid=(NN // _BN,),
        in_specs=[
            pl.BlockSpec((NC, _BN, 8), lambda i: (0, i, 0)),
            pl.BlockSpec((_BN, 3), lambda i: (i, 0)),
            _full((3, 16)), _full((1, 16)), _full((3, 16)),
        ],
        out_specs=[pl.BlockSpec((_BN, 16), lambda i: (i, 0)),
                   pl.BlockSpec((_BN, 1), lambda i: (i, 0))],
        out_shape=[jax.ShapeDtypeStruct((NN, 16), jnp.float32),
                   jax.ShapeDtypeStruct((NN, 1), jnp.float32)],
    )(P, x, W1l, b1, W1r)


def _dense2(P, h1, inv, W2l, b2, W2r, W3l):
    def body(p_ref, h1_ref, inv_ref, wl, b, wr, w3, h_ref, y_ref):
        mean = (p_ref[0] + p_ref[1]) * inv_ref[...]
        h = mean @ wl[...] + b[...] + h1_ref[...] @ wr[...]
        h = jnp.maximum(h, 0.0)
        h_ref[...] = h
        y_ref[...] = h @ w3[...]

    return pl.pallas_call(
        body,
        grid=(NN // _BN,),
        in_specs=[
            pl.BlockSpec((NC, _BN, 16), lambda i: (0, i, 0)),
            pl.BlockSpec((_BN, 16), lambda i: (i, 0)),
            pl.BlockSpec((_BN, 1), lambda i: (i, 0)),
            _full((16, 16)), _full((1, 16)), _full((16, 16)), _full((16, 8)),
        ],
        out_specs=[pl.BlockSpec((_BN, 16), lambda i: (i, 0)),
                   pl.BlockSpec((_BN, 8), lambda i: (i, 0))],
        out_shape=[jax.ShapeDtypeStruct((NN, 16), jnp.float32),
                   jax.ShapeDtypeStruct((NN, 8), jnp.float32)],
    )(P, h1, inv, W2l, b2, W2r, W3l)


def _dense3(P, h2, inv, b3, W3r):
    def body(p_ref, h2_ref, inv_ref, b, wr, o_ref):
        mean_y = (p_ref[0, :, 0:1] + p_ref[1, :, 0:1]) * inv_ref[...]
        o_ref[...] = jax.nn.sigmoid(mean_y + b[...] + h2_ref[...] @ wr[...])

    return pl.pallas_call(
        body,
        grid=(NN // _BN,),
        in_specs=[
            pl.BlockSpec((NC, _BN, 8), lambda i: (0, i, 0)),
            pl.BlockSpec((_BN, 16), lambda i: (i, 0)),
            pl.BlockSpec((_BN, 1), lambda i: (i, 0)),
            _full((1, 1)), _full((16, 1)),
        ],
        out_specs=pl.BlockSpec((_BN, 1), lambda i: (i, 0)),
        out_shape=jax.ShapeDtypeStruct((NN, 1), jnp.float32),
    )(P, h2, inv, b3, W3r)


def kernel(x, edge_index, W1l, b1, W1r, W2l, b2, W2r, W3l, b3, W3r):
    pad = EP - NE
    src = jnp.concatenate([edge_index[0], jnp.zeros((pad,), jnp.int32)])
    dst = jnp.concatenate([edge_index[1], jnp.full((pad,), NN, jnp.int32)])
    src2d = src.reshape(NROWS, LANE)
    dst2d = dst.reshape(NROWS, LANE)

    xpad = jnp.concatenate(
        [x, jnp.ones((NN, 1), jnp.float32), jnp.zeros((NN, 4), jnp.float32)],
        axis=1)
    P1 = _seg8(src2d, dst2d, xpad, jnp.zeros((N_ACC, 8), jnp.float32))
    h1, inv = _dense1(P1, x, W1l, b1.reshape(1, 16), W1r)

    P2 = _seg16(src2d, dst2d, h1, jnp.zeros((N_ACC, 16), jnp.float32))
    W3p = jnp.pad(W3l, ((0, 0), (0, 7)))
    h2, y = _dense2(P2, h1, inv, W2l, b2.reshape(1, 16), W2r, W3p)

    P3 = _seg8(src2d, dst2d, y, jnp.zeros((N_ACC, 8), jnp.float32))
    return _dense3(P3, h2, inv, b3.reshape(1, 1), W3r)


# speedup vs baseline: 49.6833x; 1.3738x over previous
import functools

import jax
import jax.numpy as jnp
from jax import lax
from jax.experimental import pallas as pl
from jax.experimental.pallas import tpu as pltpu
from jax.experimental.pallas import tpu_sc as plsc

NN = 100000
NE = 3200000
LANE = 128
NC, NS = 2, 16
NW = NC * NS
RPT = 784
NROWS = RPT * NW
EP = NROWS * LANE
N_ACC = 102400
ZR = N_ACC // NS


def _seg_sum(width, kg):
    PAIR = 2 * kg
    assert RPT % PAIR == 0
    mesh = plsc.VectorSubcoreMesh(core_axis_name="c", subcore_axis_name="s")

    @functools.partial(
        pl.kernel,
        out_type=jax.ShapeDtypeStruct((NC, N_ACC, width), jnp.float32),
        mesh=mesh,
        scratch_types=[
            pltpu.VMEM_SHARED((N_ACC, width), jnp.float32),
            pltpu.VMEM((PAIR, LANE), jnp.int32),
            pltpu.VMEM((PAIR, LANE), jnp.int32),
            pltpu.VMEM((kg, LANE, width), jnp.float32),
            pltpu.VMEM((kg, LANE, width), jnp.float32),
            pltpu.SemaphoreType.DMA,
            pltpu.SemaphoreType.DMA,
        ],
        compiler_params=pltpu.CompilerParams(use_tc_tiling_on_sc=False),
    )
    def seg(src_hbm, dst_hbm, table_hbm, zeros_hbm, out_hbm,
            accum, sbuf, dbuf, rbufa, rbufb, gsem, ssem):
        c = lax.axis_index("c")
        s = lax.axis_index("s")
        tid = s * NC + c
        pltpu.sync_copy(zeros_hbm.at[pl.ds(s * ZR, ZR)], accum.at[pl.ds(s * ZR, ZR)])
        plsc.subcore_barrier()

        row_base = tid * RPT

        def chunk(m, carry):
            r0 = row_base + m * PAIR
            pltpu.sync_copy(src_hbm.at[pl.ds(r0, PAIR)], sbuf)
            pltpu.sync_copy(dst_hbm.at[pl.ds(r0, PAIR)], dbuf)
            ga = [pltpu.async_copy(table_hbm.at[sbuf.at[j]], rbufa.at[j], gsem)
                  for j in range(kg)]
            gb = [pltpu.async_copy(table_hbm.at[sbuf.at[kg + j]], rbufb.at[j], gsem)
                  for j in range(kg)]
            for g in ga:
                g.wait()
            sa = [pltpu.async_copy(rbufa.at[j], accum.at[dbuf.at[j]], ssem, add=True)
                  for j in range(kg)]
            for g in gb:
                g.wait()
            sb = [pltpu.async_copy(rbufb.at[j], accum.at[dbuf.at[kg + j]], ssem, add=True)
                  for j in range(kg)]
            for t in sa + sb:
                t.wait()
            return carry

        lax.fori_loop(0, RPT // PAIR, chunk, 0)
        plsc.subcore_barrier()
        pltpu.sync_copy(accum.at[pl.ds(s * ZR, ZR)],
                        out_hbm.at[c, pl.ds(s * ZR, ZR)])

    return seg


_seg8 = _seg_sum(8, 7)
_seg16 = _seg_sum(16, 4)


NP = N_ACC
_BN = 25600
_R8 = _BN // 16
_R16 = _BN // 8


def _full(shape):
    return pl.BlockSpec(shape, lambda i: tuple(0 for _ in shape))


def _m8to16(W, half):
    kin = W.shape[0]
    j = jnp.arange(8)[:, None, None]
    k = jnp.arange(kin)[None, :, None]
    c = jnp.arange(16)[None, None, :]
    rows = jnp.broadcast_to((j + 8 * half) * 8 + k, (8, kin, 16))
    cols = jnp.broadcast_to(j * 16 + c, (8, kin, 16))
    vals = jnp.broadcast_to(W[None], (8, kin, 16))
    return jnp.zeros((128, 128), jnp.float32).at[rows, cols].set(vals)


def _bd16(W):
    Wp = jnp.pad(W, ((0, 0), (0, 16 - W.shape[1])))
    j = jnp.arange(8)[:, None, None]
    f = jnp.arange(16)[None, :, None]
    c = jnp.arange(16)[None, None, :]
    rows = jnp.broadcast_to(j * 16 + f, (8, 16, 16))
    cols = jnp.broadcast_to(j * 16 + c, (8, 16, 16))
    vals = jnp.broadcast_to(Wp[None], (8, 16, 16))
    return jnp.zeros((128, 128), jnp.float32).at[rows, cols].set(vals)


def _deg8():
    j = jnp.arange(16)[:, None]
    k = jnp.arange(8)[None, :]
    rows = jnp.broadcast_to(j * 8 + 3, (16, 8))
    cols = j * 8 + k
    return jnp.zeros((128, 128), jnp.float32).at[rows, cols].set(1.0)


def _pick16(field):
    W = jnp.zeros((8, 16), jnp.float32).at[field].set(1.0)
    return _m8to16(W, 0), _m8to16(W, 1)


def _t16to8():
    j = jnp.arange(8)[:, None]
    k = jnp.arange(8)[None, :]
    rows = j * 16 + k
    t0 = jnp.zeros((128, 128), jnp.float32).at[rows, j * 8 + k].set(1.0)
    t1 = jnp.zeros((128, 128), jnp.float32).at[rows, (j + 8) * 8 + k].set(1.0)
    return t0, t1


def _ilv(a, b):
    return jnp.stack([a, b], axis=1).reshape(2 * a.shape[0], 128)


def _inv16(p1s, da, db):
    deg = _ilv(p1s @ da, p1s @ db)
    return 1.0 / jnp.maximum(deg, 1.0)


def _dense1(P1p, xp, d8, ma_l, mb_l, ma_r, mb_r, bt):
    def body(p_ref, x_ref, d8_, mal, mbl, mar, mbr, b, h_ref):
        p1s = p_ref[0] + p_ref[1]
        sm = p1s * (1.0 / jnp.maximum(p1s @ d8_[...], 1.0))
        x8 = x_ref[...]
        ha = sm @ mal[...] + x8 @ mar[...] + b[...]
        hb = sm @ mbl[...] + x8 @ mbr[...] + b[...]
        h_ref[...] = _ilv(jnp.maximum(ha, 0.0), jnp.maximum(hb, 0.0))

    return pl.pallas_call(
        body,
        grid=(NP // _BN,),
        in_specs=[
            pl.BlockSpec((NC, _R8, 128), lambda i: (0, i, 0)),
            pl.BlockSpec((_R8, 128), lambda i: (i, 0)),
        ] + [_full((128, 128))] * 5 + [_full((1, 128))],
        out_specs=pl.BlockSpec((_R16, 128), lambda i: (i, 0)),
        out_shape=jax.ShapeDtypeStruct((NP * 16 // 128, 128), jnp.float32),
    )(P1p, xp, d8, ma_l, mb_l, ma_r, mb_r, bt)


def _dense2(P2p, h1p, P1p, da, db, bw_l, bw_r, bw3, t0, t1, bt):
    def body(p_ref, h1_ref, p1_ref, da_, db_, wl, wr, w3, t0_, t1_, b,
             h_ref, y_ref):
        inv = _inv16(p1_ref[0] + p1_ref[1], da_[...], db_[...])
        s2m = (p_ref[0] + p_ref[1]) * inv
        h = jnp.maximum(s2m @ wl[...] + h1_ref[...] @ wr[...] + b[...], 0.0)
        h_ref[...] = h
        y16 = h @ w3[...]
        ye = y16.reshape(_R8, 2, 128)[:, 0, :]
        yo = y16.reshape(_R8, 2, 128)[:, 1, :]
        y_ref[...] = ye @ t0_[...] + yo @ t1_[...]

    return pl.pallas_call(
        body,
        grid=(NP // _BN,),
        in_specs=[
            pl.BlockSpec((NC, _R16, 128), lambda i: (0, i, 0)),
            pl.BlockSpec((_R16, 128), lambda i: (i, 0)),
            pl.BlockSpec((NC, _R8, 128), lambda i: (0, i, 0)),
        ] + [_full((128, 128))] * 7 + [_full((1, 128))],
        out_specs=[pl.BlockSpec((_R16, 128), lambda i: (i, 0)),
                   pl.BlockSpec((_R8, 128), lambda i: (i, 0))],
        out_shape=[jax.ShapeDtypeStruct((NP * 16 // 128, 128), jnp.float32),
                   jax.ShapeDtypeStruct((NP * 8 // 128, 128), jnp.float32)],
    )(P2p, h1p, P1p, da, db, bw_l, bw_r, bw3, t0, t1, bt)


def _dense3(P3p, h2p, P1p, da, db, fa, fb, bwr, b3):
    def body(p_ref, h2_ref, p1_ref, da_, db_, fa_, fb_, wr, b, o_ref):
        inv = _inv16(p1_ref[0] + p1_ref[1], da_[...], db_[...])
        p3s = p_ref[0] + p_ref[1]
        my = _ilv(p3s @ fa_[...], p3s @ fb_[...]) * inv
        o_ref[...] = jax.nn.sigmoid(my + h2_ref[...] @ wr[...] + b[0, 0])

    return pl.pallas_call(
        body,
        grid=(NP // _BN,),
        in_specs=[
            pl.BlockSpec((NC, _R8, 128), lambda i: (0, i, 0)),
            pl.BlockSpec((_R16, 128), lambda i: (i, 0)),
            pl.BlockSpec((NC, _R8, 128), lambda i: (0, i, 0)),
        ] + [_full((128, 128))] * 5 + [_full((1, 1))],
        out_specs=pl.BlockSpec((_R16, 128), lambda i: (i, 0)),
        out_shape=jax.ShapeDtypeStruct((NP * 16 // 128, 128), jnp.float32),
    )(P3p, h2p, P1p, da, db, fa, fb, bwr, b3)


def kernel(x, edge_index, W1l, b1, W1r, W2l, b2, W2r, W3l, b3, W3r):
    pad = EP - NE
    src = jnp.concatenate([edge_index[0], jnp.zeros((pad,), jnp.int32)])
    dst = jnp.concatenate([edge_index[1], jnp.full((pad,), NN, jnp.int32)])
    src2d = src.reshape(NROWS, LANE)
    dst2d = dst.reshape(NROWS, LANE)

    d8 = _deg8()
    da, db = _pick16(3)
    fa, fb = _pick16(0)
    t0, t1 = _t16to8()
    bt1 = jnp.tile(b1.reshape(1, 16), (1, 8))
    bt2 = jnp.tile(b2.reshape(1, 16), (1, 8))

    xpad = jnp.concatenate(
        [x, jnp.ones((NN, 1), jnp.float32), jnp.zeros((NN, 4), jnp.float32)],
        axis=1)
    xp = jnp.pad(xpad, ((0, NP - NN), (0, 0))).reshape(NP * 8 // 128, 128)
    P1 = _seg8(src2d, dst2d, xp.reshape(NP, 8),
               jnp.zeros((N_ACC, 8), jnp.float32))
    P1p = P1.reshape(NC, N_ACC * 8 // 128, 128)
    h1p = _dense1(P1p, xp, d8, _m8to16(W1l, 0), _m8to16(W1l, 1),
                  _m8to16(W1r, 0), _m8to16(W1r, 1), bt1)

    P2 = _seg16(src2d, dst2d, h1p.reshape(NP, 16),
                jnp.zeros((N_ACC, 16), jnp.float32))
    P2p = P2.reshape(NC, N_ACC * 16 // 128, 128)
    h2p, yp = _dense2(P2p, h1p, P1p, da, db, _bd16(W2l), _bd16(W2r),
                      _bd16(W3l), t0, t1, bt2)

    P3 = _seg8(src2d, dst2d, yp.reshape(NP, 8),
               jnp.zeros((N_ACC, 8), jnp.float32))
    P3p = P3.reshape(NC, N_ACC * 8 // 128, 128)
    op = _dense3(P3p, h2p, P1p, da, db, fa, fb, _bd16(W3r),
                 b3.reshape(1, 1))
    return op.reshape(NP, 16)[:NN, 0:1]


# speedup vs baseline: 51.5399x; 1.0374x over previous
import functools

import jax
import jax.numpy as jnp
from jax import lax
from jax.experimental import pallas as pl
from jax.experimental.pallas import tpu as pltpu
from jax.experimental.pallas import tpu_sc as plsc

NN = 100000
NE = 3200000
LANE = 128
NC, NS = 2, 16
NW = NC * NS
RPT = 784
NROWS = RPT * NW
EP = NROWS * LANE
N_ACC = 102400
ZR = N_ACC // NS


def _seg_sum(width, kg):
    PAIR = 2 * kg
    assert RPT % PAIR == 0
    mesh = plsc.VectorSubcoreMesh(core_axis_name="c", subcore_axis_name="s")

    @functools.partial(
        pl.kernel,
        out_type=jax.ShapeDtypeStruct((NC, N_ACC, width), jnp.float32),
        mesh=mesh,
        scratch_types=[
            pltpu.VMEM_SHARED((N_ACC, width), jnp.float32),
            pltpu.VMEM((PAIR, LANE), jnp.int32),
            pltpu.VMEM((PAIR, LANE), jnp.int32),
            pltpu.VMEM((kg, LANE, width), jnp.float32),
            pltpu.VMEM((kg, LANE, width), jnp.float32),
            pltpu.SemaphoreType.DMA,
            pltpu.SemaphoreType.DMA,
        ],
        compiler_params=pltpu.CompilerParams(use_tc_tiling_on_sc=False),
    )
    def seg(src_hbm, dst_hbm, table_hbm, zeros_hbm, out_hbm,
            accum, sbuf, dbuf, rbufa, rbufb, gsem, ssem):
        c = lax.axis_index("c")
        s = lax.axis_index("s")
        tid = s * NC + c
        pltpu.sync_copy(zeros_hbm.at[pl.ds(s * ZR, ZR)], accum.at[pl.ds(s * ZR, ZR)])
        plsc.subcore_barrier()

        row_base = tid * RPT

        def chunk(m, carry):
            r0 = row_base + m * PAIR
            pltpu.sync_copy(src_hbm.at[pl.ds(r0, PAIR)], sbuf)
            pltpu.sync_copy(dst_hbm.at[pl.ds(r0, PAIR)], dbuf)
            ga = [pltpu.async_copy(table_hbm.at[sbuf.at[j]], rbufa.at[j], gsem)
                  for j in range(kg)]
            gb = [pltpu.async_copy(table_hbm.at[sbuf.at[kg + j]], rbufb.at[j], gsem)
                  for j in range(kg)]
            for g in ga:
                g.wait()
            sa = [pltpu.async_copy(rbufa.at[j], accum.at[dbuf.at[j]], ssem, add=True)
                  for j in range(kg)]
            for g in gb:
                g.wait()
            sb = [pltpu.async_copy(rbufb.at[j], accum.at[dbuf.at[kg + j]], ssem, add=True)
                  for j in range(kg)]
            for t in sa + sb:
                t.wait()
            return carry

        lax.fori_loop(0, RPT // PAIR, chunk, 0)
        plsc.subcore_barrier()
        pltpu.sync_copy(accum.at[pl.ds(s * ZR, ZR)],
                        out_hbm.at[c, pl.ds(s * ZR, ZR)])

    return seg


_seg8 = _seg_sum(8, 7)
_seg16 = _seg_sum(16, 4)


NP = N_ACC
_BN = 25600
_R8 = _BN // 16
_R16 = _BN // 8


def _full(shape):
    return pl.BlockSpec(shape, lambda i: tuple(0 for _ in shape))


def _m8to16(W, half):
    Wp = jnp.pad(W.astype(jnp.float32), ((0, 8 - W.shape[0]), (0, 0)))
    return jnp.kron(jnp.eye(16, 8, -8 * half, dtype=jnp.float32), Wp)


def _bd16(W):
    Wp = jnp.pad(W.astype(jnp.float32), ((0, 0), (0, 16 - W.shape[1])))
    return jnp.kron(jnp.eye(8, dtype=jnp.float32), Wp)


def _deg8():
    u = jnp.broadcast_to((jnp.arange(8) == 3).astype(jnp.float32)[:, None],
                         (8, 8))
    return jnp.kron(jnp.eye(16, dtype=jnp.float32), u)


def _pick16(field):
    W = jnp.broadcast_to((jnp.arange(8) == field).astype(jnp.float32)[:, None],
                         (8, 16))
    return _m8to16(W, 0), _m8to16(W, 1)


def _t16to8():
    v = jnp.eye(16, 8, dtype=jnp.float32)
    t0 = jnp.kron(jnp.eye(8, 16, 0, dtype=jnp.float32), v)
    t1 = jnp.kron(jnp.eye(8, 16, 8, dtype=jnp.float32), v)
    return t0, t1


def _ilv(a, b):
    return jnp.stack([a, b], axis=1).reshape(2 * a.shape[0], 128)


def _inv16(p1s, da, db):
    deg = _ilv(p1s @ da, p1s @ db)
    return 1.0 / jnp.maximum(deg, 1.0)


def _dense1(P1p, xp, d8, ma_l, mb_l, ma_r, mb_r, bt):
    def body(p_ref, x_ref, d8_, mal, mbl, mar, mbr, b, h_ref):
        p1s = p_ref[0] + p_ref[1]
        sm = p1s * (1.0 / jnp.maximum(p1s @ d8_[...], 1.0))
        x8 = x_ref[...]
        ha = sm @ mal[...] + x8 @ mar[...] + b[...]
        hb = sm @ mbl[...] + x8 @ mbr[...] + b[...]
        h_ref[...] = _ilv(jnp.maximum(ha, 0.0), jnp.maximum(hb, 0.0))

    return pl.pallas_call(
        body,
        grid=(NP // _BN,),
        in_specs=[
            pl.BlockSpec((NC, _R8, 128), lambda i: (0, i, 0)),
            pl.BlockSpec((_R8, 128), lambda i: (i, 0)),
        ] + [_full((128, 128))] * 5 + [_full((1, 128))],
        out_specs=pl.BlockSpec((_R16, 128), lambda i: (i, 0)),
        out_shape=jax.ShapeDtypeStruct((NP * 16 // 128, 128), jnp.float32),
    )(P1p, xp, d8, ma_l, mb_l, ma_r, mb_r, bt)


def _dense2(P2p, h1p, P1p, da, db, bw_l, bw_r, bw3, t0, t1, bt):
    def body(p_ref, h1_ref, p1_ref, da_, db_, wl, wr, w3, t0_, t1_, b,
             h_ref, y_ref):
        inv = _inv16(p1_ref[0] + p1_ref[1], da_[...], db_[...])
        s2m = (p_ref[0] + p_ref[1]) * inv
        h = jnp.maximum(s2m @ wl[...] + h1_ref[...] @ wr[...] + b[...], 0.0)
        h_ref[...] = h
        y16 = h @ w3[...]
        ye = y16.reshape(_R8, 2, 128)[:, 0, :]
        yo = y16.reshape(_R8, 2, 128)[:, 1, :]
        y_ref[...] = ye @ t0_[...] + yo @ t1_[...]

    return pl.pallas_call(
        body,
        grid=(NP // _BN,),
        in_specs=[
            pl.BlockSpec((NC, _R16, 128), lambda i: (0, i, 0)),
            pl.BlockSpec((_R16, 128), lambda i: (i, 0)),
            pl.BlockSpec((NC, _R8, 128), lambda i: (0, i, 0)),
        ] + [_full((128, 128))] * 7 + [_full((1, 128))],
        out_specs=[pl.BlockSpec((_R16, 128), lambda i: (i, 0)),
                   pl.BlockSpec((_R8, 128), lambda i: (i, 0))],
        out_shape=[jax.ShapeDtypeStruct((NP * 16 // 128, 128), jnp.float32),
                   jax.ShapeDtypeStruct((NP * 8 // 128, 128), jnp.float32)],
    )(P2p, h1p, P1p, da, db, bw_l, bw_r, bw3, t0, t1, bt)


def _dense3(P3p, h2p, P1p, da, db, fa, fb, bwr, b3):
    def body(p_ref, h2_ref, p1_ref, da_, db_, fa_, fb_, wr, b, o_ref):
        inv = _inv16(p1_ref[0] + p1_ref[1], da_[...], db_[...])
        p3s = p_ref[0] + p_ref[1]
        my = _ilv(p3s @ fa_[...], p3s @ fb_[...]) * inv
        o_ref[...] = jax.nn.sigmoid(my + h2_ref[...] @ wr[...] + b[0, 0])

    return pl.pallas_call(
        body,
        grid=(NP // _BN,),
        in_specs=[
            pl.BlockSpec((NC, _R8, 128), lambda i: (0, i, 0)),
            pl.BlockSpec((_R16, 128), lambda i: (i, 0)),
            pl.BlockSpec((NC, _R8, 128), lambda i: (0, i, 0)),
        ] + [_full((128, 128))] * 5 + [_full((1, 1))],
        out_specs=pl.BlockSpec((_R16, 128), lambda i: (i, 0)),
        out_shape=jax.ShapeDtypeStruct((NP * 16 // 128, 128), jnp.float32),
    )(P3p, h2p, P1p, da, db, fa, fb, bwr, b3)


def kernel(x, edge_index, W1l, b1, W1r, W2l, b2, W2r, W3l, b3, W3r):
    pad = EP - NE
    src = jnp.concatenate([edge_index[0], jnp.zeros((pad,), jnp.int32)])
    dst = jnp.concatenate([edge_index[1], jnp.full((pad,), NN, jnp.int32)])
    src2d = src.reshape(NROWS, LANE)
    dst2d = dst.reshape(NROWS, LANE)

    d8 = _deg8()
    da, db = _pick16(3)
    fa, fb = _pick16(0)
    t0, t1 = _t16to8()
    bt1 = jnp.tile(b1.reshape(1, 16), (1, 8))
    bt2 = jnp.tile(b2.reshape(1, 16), (1, 8))

    xpad = jnp.concatenate(
        [x, jnp.ones((NN, 1), jnp.float32), jnp.zeros((NN, 4), jnp.float32)],
        axis=1)
    xp = jnp.pad(xpad, ((0, NP - NN), (0, 0))).reshape(NP * 8 // 128, 128)
    P1 = _seg8(src2d, dst2d, xp.reshape(NP, 8),
               jnp.zeros((N_ACC, 8), jnp.float32))
    P1p = P1.reshape(NC, N_ACC * 8 // 128, 128)
    h1p = _dense1(P1p, xp, d8, _m8to16(W1l, 0), _m8to16(W1l, 1),
                  _m8to16(W1r, 0), _m8to16(W1r, 1), bt1)

    P2 = _seg16(src2d, dst2d, h1p.reshape(NP, 16),
                jnp.zeros((N_ACC, 16), jnp.float32))
    P2p = P2.reshape(NC, N_ACC * 16 // 128, 128)
    h2p, yp = _dense2(P2p, h1p, P1p, da, db, _bd16(W2l), _bd16(W2r),
                      _bd16(W3l), t0, t1, bt2)

    P3 = _seg8(src2d, dst2d, yp.reshape(NP, 8),
               jnp.zeros((N_ACC, 8), jnp.float32))
    P3p = P3.reshape(NC, N_ACC * 8 // 128, 128)
    op = _dense3(P3p, h2p, P1p, da, db, fa, fb, _bd16(W3r),
                 b3.reshape(1, 1))
    return op[:, ::16].reshape(NP)[:NN].reshape(NN, 1)
